# Initial kernel scaffold; baseline (speedup 1.0000x reference)
import jax
import jax.numpy as jnp
from jax.experimental import pallas as pl


def kernel(x, W):
    raise NotImplementedError("write your pallas kernel here")


# speedup vs baseline: 5.1153x; 5.1153x over previous
import functools

import jax
import jax.numpy as jnp
from jax import lax
from jax.experimental import pallas as pl
from jax.experimental.pallas import tpu as pltpu
from jax.experimental.pallas import tpu_sc as plsc

_NC = 2
_NS = 16
_NW = _NC * _NS
_L = 16

_D = 4
_CHUNK = 5120


def _body(x_ref, w_ref, out_ref, wt_v, idx_v, out_v, n_chunks):
    wid = lax.axis_index("s") * _NC + lax.axis_index("c")

    pltpu.sync_copy(w_ref, wt_v)
    for k in range(3):
        wt_v[pl.ds(k * _L, _L)] = wt_v[pl.ds(k * _L, _L)] * 0.5 + 1.0

    lanes4 = lax.iota(jnp.int32, _L) * _D
    per_tile = n_chunks * _CHUNK
    base_t = wid * per_tile

    def chunk_body(ci, _):
        cbase = base_t + ci * _CHUNK
        pltpu.sync_copy(x_ref.at[pl.ds(cbase, _CHUNK)], idx_v)

        def inner(i, _):
            idx16 = idx_v[pl.ds(i * _L, _L)]
            off = idx16 * _D
            sbase = lanes4 + i * (_L * _D)
            for c in range(_D):
                g = plsc.load_gather(wt_v, [off + c])
                plsc.store_scatter(out_v, [sbase + c], g)
            return 0

        lax.fori_loop(0, _CHUNK // _L, inner, 0)
        pltpu.sync_copy(out_v, out_ref.at[pl.ds(cbase * _D, _CHUNK * _D)])
        return 0

    lax.fori_loop(0, n_chunks, chunk_body, 0)


@functools.partial(jax.jit, static_argnames=("n",))
def _lookup(xf, wf, n):
    n_chunks = n // (_NW * _CHUNK)
    mesh = plsc.VectorSubcoreMesh(core_axis_name="c", subcore_axis_name="s")
    run = pl.kernel(
        functools.partial(_body, n_chunks=n_chunks),
        out_type=jax.ShapeDtypeStruct((n * _D,), jnp.float32),
        mesh=mesh,
        scratch_types=[
            pltpu.VMEM((3 * _L,), jnp.float32),
            pltpu.VMEM((_CHUNK,), jnp.int32),
            pltpu.VMEM((_CHUNK * _D,), jnp.float32),
        ],
        compiler_params=pltpu.CompilerParams(needs_layout_passes=False),
    )
    return run(xf, wf)


def kernel(x, W):
    b, s = x.shape
    n = b * s
    xf = x.reshape(-1).astype(jnp.int32)
    wf = jnp.pad(W.reshape(-1).astype(jnp.float32), (0, 3 * _L - W.size))
    out = _lookup(xf, wf, n)
    return out.reshape(b, s, _D)


# speedup vs baseline: 5.4802x; 1.0713x over previous
import functools

import jax
import jax.numpy as jnp
from jax import lax
from jax.experimental import pallas as pl
from jax.experimental.pallas import tpu as pltpu
from jax.experimental.pallas import tpu_sc as plsc

_NC = 2
_NS = 16
_NW = _NC * _NS
_L = 16

_D = 4
_CHUNK = 5120


def _body(x_ref, w_ref, out_ref, wt_v, i0, i1, o0, o1, si0, si1, so0, so1,
          n_chunks):
    wid = lax.axis_index("s") * _NC + lax.axis_index("c")

    pltpu.sync_copy(w_ref, wt_v)
    for k in range(3):
        wt_v[pl.ds(k * _L, _L)] = wt_v[pl.ds(k * _L, _L)] * 0.5 + 1.0

    lanes4 = lax.iota(jnp.int32, _L) * _D
    per_tile = n_chunks * _CHUNK
    base_t = wid * per_tile

    ibufs = (i0, i1)
    obufs = (o0, o1)
    isems = (si0, si1)
    osems = (so0, so1)

    def in_copy(c, b):
        return pltpu.make_async_copy(
            x_ref.at[pl.ds(base_t + c * _CHUNK, _CHUNK)], ibufs[b], isems[b])

    def out_copy(c, b):
        return pltpu.make_async_copy(
            obufs[b], out_ref.at[pl.ds((base_t + c * _CHUNK) * _D, _CHUNK * _D)],
            osems[b])

    def compute(b):
        iv = ibufs[b]
        ov = obufs[b]

        @plsc.parallel_loop(0, _CHUNK // _L, unroll=8)
        def _(i):
            idx16 = iv[pl.ds(i * _L, _L)]
            off = idx16 * _D
            sbase = lanes4 + i * (_L * _D)
            for c in range(_D):
                g = plsc.load_gather(wt_v, [off + c])
                plsc.store_scatter(ov, [sbase + c], g)

    in_copy(0, 0).start()
    in_copy(1, 1).start()

    def superstep(s, _):
        for b in range(2):
            c = 2 * s + b
            in_copy(c, b).wait()

            @pl.when(s > 0)
            def _():
                out_copy(c - 2, b).wait()

            compute(b)
            out_copy(c, b).start()

            @pl.when(c + 2 < n_chunks)
            def _():
                in_copy(c + 2, b).start()

        return 0

    lax.fori_loop(0, n_chunks // 2, superstep, 0)
    out_copy(n_chunks - 2, 0).wait()
    out_copy(n_chunks - 1, 1).wait()


@functools.partial(jax.jit, static_argnames=("n",))
def _lookup(xf, wf, n):
    n_chunks = n // (_NW * _CHUNK)
    mesh = plsc.VectorSubcoreMesh(core_axis_name="c", subcore_axis_name="s")
    run = pl.kernel(
        functools.partial(_body, n_chunks=n_chunks),
        out_type=jax.ShapeDtypeStruct((n * _D,), jnp.float32),
        mesh=mesh,
        scratch_types=[
            pltpu.VMEM((3 * _L,), jnp.float32),
            pltpu.VMEM((_CHUNK,), jnp.int32),
            pltpu.VMEM((_CHUNK,), jnp.int32),
            pltpu.VMEM((_CHUNK * _D,), jnp.float32),
            pltpu.VMEM((_CHUNK * _D,), jnp.float32),
            pltpu.SemaphoreType.DMA,
            pltpu.SemaphoreType.DMA,
            pltpu.SemaphoreType.DMA,
            pltpu.SemaphoreType.DMA,
        ],
        compiler_params=pltpu.CompilerParams(needs_layout_passes=False),
    )
    return run(xf, wf)


def kernel(x, W):
    b, s = x.shape
    n = b * s
    xf = x.reshape(-1).astype(jnp.int32)
    wf = jnp.pad(W.reshape(-1).astype(jnp.float32), (0, 3 * _L - W.size))
    out = _lookup(xf, wf, n)
    return out.reshape(b, s, _D)
